# baseline (speedup 1.0000x reference)
import functools

import jax
import jax.numpy as jnp
from jax import lax
from jax.experimental import pallas as pl
from jax.experimental.pallas import tpu as pltpu
from jax.experimental.pallas import tpu_sc as plsc

_FEATURE_DIMS = [119, 5, 12, 12, 10, 6, 6, 2, 2]
_DIM = 128
_N = 100000

_NC, _NS = 2, 16
_NW = _NC * _NS
_CB = 3200
_NPAD = _NW * _CB
_B = 320
_NCH = _CB // _B
_K = 4
_KP = _K // 2
_ROWS = 119 + 5 * 12 * 2 + 12 * 10 * 2 + 6 * 6
_D2 = _DIM // 2


def _sc_body(idx_hbm, tab_hbm, out_hbm, idx_v, tab_v, out_a, out_b,
             sem_a, sem_b):
    wid = lax.axis_index("s") * _NC + lax.axis_index("c")
    pltpu.sync_copy(tab_hbm, tab_v)
    pltpu.sync_copy(idx_hbm.at[wid], idx_v)

    mask_hi = jnp.full((16,), -65536, dtype=jnp.int32)

    def run_chunk(ch, out_v):
        @plsc.parallel_loop(0, _B // 16, unroll=2)
        def group_body(g):
            goff = ch * _B + g * 16
            rv = [idx_v[j, pl.ds(goff, 16)] for j in range(_KP)]
            for m in range(16):
                rows = []
                for j in range(_KP):
                    e = rv[j][m]
                    rows.append(e & 0xFFFF)
                    rows.append(e >> 16)
                ob = (g * 16 + m) * _DIM
                for b in range(_D2 // 16):
                    acc = plsc.bitcast(
                        tab_v[pl.ds(rows[0] + b * 16, 16)], jnp.bfloat16)
                    for j in range(1, _K):
                        acc = acc + plsc.bitcast(
                            tab_v[pl.ds(rows[j] + b * 16, 16)], jnp.bfloat16)
                    w = plsc.bitcast(acc, jnp.int32)
                    lo = plsc.bitcast(w << 16, jnp.float32)
                    hi = plsc.bitcast(w & mask_hi, jnp.float32)
                    out_v[pl.ds(ob + b * 16, 16)] = lo
                    out_v[pl.ds(ob + 64 + b * 16, 16)] = hi

    def pair_body(p, _):
        for b, (buf, sem) in enumerate(((out_a, sem_a), (out_b, sem_b))):
            ch = 2 * p + b

            @pl.when(p > 0)
            def _():
                pltpu.make_async_copy(
                    buf, out_hbm.at[pl.ds(0, _B * _DIM)], sem).wait()

            run_chunk(ch, buf)
            pltpu.async_copy(
                buf,
                out_hbm.at[pl.ds((wid * _CB + ch * _B) * _DIM, _B * _DIM)],
                sem)
        return 0

    lax.fori_loop(0, _NCH // 2, pair_body, 0)
    for buf, sem in ((out_a, sem_a), (out_b, sem_b)):
        pltpu.make_async_copy(
            buf, out_hbm.at[pl.ds(0, _B * _DIM)], sem).wait()


def kernel(node_feat, emb_0, emb_1, emb_2, emb_3, emb_4, emb_5, emb_6,
           emb_7, emb_8):
    tab1 = (emb_1[:, None, None, :] + emb_2[None, :, None, :]
            + emb_8[None, None, :, :]).reshape(120, _DIM)
    tab2 = (emb_3[:, None, None, :] + emb_4[None, :, None, :]
            + emb_7[None, None, :, :]).reshape(240, _DIM)
    tab3 = (emb_5[:, None, :] + emb_6[None, :, :]).reshape(36, _DIM)
    tab = jnp.concatenate([emb_0, tab1, tab2, tab3], axis=0)
    tu = jax.lax.bitcast_convert_type(
        tab.astype(jnp.bfloat16), jnp.uint16).astype(jnp.uint32)
    tabp = jax.lax.bitcast_convert_type(
        tu[:, :_D2] | (tu[:, _D2:] << 16), jnp.int32)
    f = node_feat.astype(jnp.int32)
    r0 = f[:, 0]
    r1 = 119 + (f[:, 1] * 12 + f[:, 2]) * 2 + f[:, 8]
    r2 = 239 + (f[:, 3] * 10 + f[:, 4]) * 2 + f[:, 7]
    r3 = 479 + f[:, 5] * 6 + f[:, 6]
    idx = jnp.stack([(r0 * _D2) | ((r1 * _D2) << 16),
                     (r3 * _D2) | ((r2 * _D2) << 16)], axis=1)
    idx = jnp.pad(idx, ((0, _NPAD - _N), (0, 0)))
    idx = idx.T.reshape(_KP, _NW, _CB).transpose(1, 0, 2)

    grid_kernel = functools.partial(
        pl.kernel,
        out_type=jax.ShapeDtypeStruct((_NPAD * _DIM,), jnp.float32),
        mesh=plsc.VectorSubcoreMesh(core_axis_name="c", subcore_axis_name="s"),
        compiler_params=pltpu.CompilerParams(needs_layout_passes=False),
        scratch_types=[
            pltpu.VMEM((_KP, _CB), jnp.int32),
            pltpu.VMEM((_ROWS * _D2,), jnp.int32),
            pltpu.VMEM((_B * _DIM,), jnp.float32),
            pltpu.VMEM((_B * _DIM,), jnp.float32),
            pltpu.SemaphoreType.DMA,
            pltpu.SemaphoreType.DMA,
        ],
    )
    out = grid_kernel(_sc_body)(idx, tabp.reshape(-1))
    return out.reshape(_NPAD, _DIM)[:_N]


# speedup vs baseline: 1.5513x; 1.5513x over previous
import functools

import jax
import jax.numpy as jnp
from jax import lax
from jax.experimental import pallas as pl
from jax.experimental.pallas import tpu as pltpu
from jax.experimental.pallas import tpu_sc as plsc

_FEATURE_DIMS = [119, 5, 12, 12, 10, 6, 6, 2, 2]
_DIM = 128
_N = 100000

_NC, _NS = 2, 16
_NW = _NC * _NS
_CB = 3200
_NPAD = _NW * _CB
_B = 320
_NCH = _CB // _B
_K = 4
_KP = _K // 2
_ROWS = 119 + 5 * 12 * 2 + 12 * 10 * 2 + 6 * 6
_D2 = _DIM // 2


def _sc_body(idx_hbm, tab_hbm, out_hbm, idx_v, tab_v, out_a, out_b,
             sem_a, sem_b):
    wid = lax.axis_index("s") * _NC + lax.axis_index("c")
    pltpu.sync_copy(tab_hbm, tab_v)
    pltpu.sync_copy(idx_hbm.at[wid], idx_v)

    mask_hi = jnp.full((16,), -65536, dtype=jnp.int32)

    def run_chunk(ch, out_v):
        @plsc.parallel_loop(0, _B // 16)
        def group_body(g):
            goff = ch * _B + g * 16
            rv = [idx_v[j, pl.ds(goff, 16)] for j in range(_KP)]
            for m in range(16):
                rows = []
                for j in range(_KP):
                    e = rv[j][m]
                    rows.append(e & 0xFFFF)
                    rows.append(e >> 16)
                ob = (g * 16 + m) * _DIM
                for b in range(_D2 // 16):
                    acc = plsc.bitcast(
                        tab_v[pl.ds(rows[0] + b * 16, 16)], jnp.bfloat16)
                    for j in range(1, _K):
                        acc = acc + plsc.bitcast(
                            tab_v[pl.ds(rows[j] + b * 16, 16)], jnp.bfloat16)
                    w = plsc.bitcast(acc, jnp.int32)
                    lo = plsc.bitcast(w << 16, jnp.float32)
                    hi = plsc.bitcast(w & mask_hi, jnp.float32)
                    out_v[pl.ds(ob + b * 16, 16)] = lo
                    out_v[pl.ds(ob + 64 + b * 16, 16)] = hi

    def pair_body(p, _):
        for b, (buf, sem) in enumerate(((out_a, sem_a), (out_b, sem_b))):
            ch = 2 * p + b

            @pl.when(p > 0)
            def _():
                pltpu.make_async_copy(
                    buf, out_hbm.at[pl.ds(0, _B * _DIM)], sem).wait()

            run_chunk(ch, buf)
            pltpu.async_copy(
                buf,
                out_hbm.at[pl.ds((wid * _CB + ch * _B) * _DIM, _B * _DIM)],
                sem)
        return 0

    lax.fori_loop(0, _NCH // 2, pair_body, 0)
    for buf, sem in ((out_a, sem_a), (out_b, sem_b)):
        pltpu.make_async_copy(
            buf, out_hbm.at[pl.ds(0, _B * _DIM)], sem).wait()


def kernel(node_feat, emb_0, emb_1, emb_2, emb_3, emb_4, emb_5, emb_6,
           emb_7, emb_8):
    tab1 = (emb_1[:, None, None, :] + emb_2[None, :, None, :]
            + emb_8[None, None, :, :]).reshape(120, _DIM)
    tab2 = (emb_3[:, None, None, :] + emb_4[None, :, None, :]
            + emb_7[None, None, :, :]).reshape(240, _DIM)
    tab3 = (emb_5[:, None, :] + emb_6[None, :, :]).reshape(36, _DIM)
    tab = jnp.concatenate([emb_0, tab1, tab2, tab3], axis=0)
    tu = jax.lax.bitcast_convert_type(
        tab.astype(jnp.bfloat16), jnp.uint16).astype(jnp.uint32)
    tabp = jax.lax.bitcast_convert_type(
        tu[:, :_D2] | (tu[:, _D2:] << 16), jnp.int32)
    f = node_feat.astype(jnp.int32)
    r0 = f[:, 0]
    r1 = 119 + (f[:, 1] * 12 + f[:, 2]) * 2 + f[:, 8]
    r2 = 239 + (f[:, 3] * 10 + f[:, 4]) * 2 + f[:, 7]
    r3 = 479 + f[:, 5] * 6 + f[:, 6]
    idx = jnp.stack([(r0 * _D2) | ((r1 * _D2) << 16),
                     (r3 * _D2) | ((r2 * _D2) << 16)], axis=1)
    idx = jnp.pad(idx, ((0, _NPAD - _N), (0, 0)))
    idx = idx.T.reshape(_KP, _NW, _CB).transpose(1, 0, 2)

    grid_kernel = functools.partial(
        pl.kernel,
        out_type=jax.ShapeDtypeStruct((_NPAD * _DIM,), jnp.float32),
        mesh=plsc.VectorSubcoreMesh(core_axis_name="c", subcore_axis_name="s"),
        compiler_params=pltpu.CompilerParams(needs_layout_passes=False),
        scratch_types=[
            pltpu.VMEM((_KP, _CB), jnp.int32),
            pltpu.VMEM((_ROWS * _D2,), jnp.int32),
            pltpu.VMEM((_B * _DIM,), jnp.float32),
            pltpu.VMEM((_B * _DIM,), jnp.float32),
            pltpu.SemaphoreType.DMA,
            pltpu.SemaphoreType.DMA,
        ],
    )
    out = grid_kernel(_sc_body)(idx, tabp.reshape(-1))
    return out.reshape(_NPAD, _DIM)[:_N]


# speedup vs baseline: 2.0628x; 1.3297x over previous
import functools

import jax
import jax.numpy as jnp
from jax import lax
from jax.experimental import pallas as pl
from jax.experimental.pallas import tpu as pltpu
from jax.experimental.pallas import tpu_sc as plsc

_FEATURE_DIMS = [119, 5, 12, 12, 10, 6, 6, 2, 2]
_DIM = 128
_N = 100000

_NC, _NS = 2, 16
_NW = _NC * _NS
_CB = 3200
_NPAD = _NW * _CB
_B = 320
_NCH = _CB // _B
_K = 4
_KP = _K // 2
_ROWS = 119 + 5 * 12 * 2 + 12 * 10 * 2 + 6 * 6
_D2 = _DIM // 2


def _sc_body(idx_hbm, tab_hbm, out_hbm, idx_v, tab_v, out_a, out_b,
             sem_a, sem_b):
    wid = lax.axis_index("s") * _NC + lax.axis_index("c")
    pltpu.sync_copy(tab_hbm, tab_v)
    pltpu.sync_copy(idx_hbm.at[wid], idx_v)

    mask_hi = jnp.full((16,), -65536, dtype=jnp.int32)

    def run_chunk(ch, out_v):
        @plsc.parallel_loop(0, _B // 16)
        def group_body(g):
            goff = ch * _B + g * 16
            rv = [idx_v[j, pl.ds(goff, 16)] for j in range(_KP)]
            for m in range(16):
                rows = []
                for j in range(_KP):
                    e = rv[j][m]
                    rows.append(e & 0xFFFF)
                    rows.append(e >> 16)
                ob = (g * 16 + m) * _DIM
                for b in range(_D2 // 16):
                    acc = plsc.bitcast(
                        tab_v[pl.ds(rows[0] + b * 16, 16)], jnp.bfloat16)
                    for j in range(1, _K):
                        acc = acc + plsc.bitcast(
                            tab_v[pl.ds(rows[j] + b * 16, 16)], jnp.bfloat16)
                    w = plsc.bitcast(acc, jnp.int32)
                    lo = plsc.bitcast(w << 16, jnp.float32)
                    hi = plsc.bitcast(w & mask_hi, jnp.float32)
                    out_v[pl.ds(ob + b * 16, 16)] = lo
                    out_v[pl.ds(ob + 64 + b * 16, 16)] = hi

    _H = _B // 2

    def start_copy(ch, buf, sem):
        s = wid * _CB + ch * _B

        @pl.when(s + _B <= _N)
        def _():
            pltpu.async_copy(
                buf, out_hbm.at[pl.ds(s * _DIM, _B * _DIM)], sem)

        @pl.when(jnp.logical_and(s < _N, s + _B > _N))
        def _():
            for _i in range(2):
                pltpu.async_copy(
                    buf.at[pl.ds(0, _H * _DIM)],
                    out_hbm.at[pl.ds(s * _DIM, _H * _DIM)], sem)

    def drain_copy(ch, buf, sem):
        @pl.when(wid * _CB + ch * _B < _N)
        def _():
            pltpu.make_async_copy(
                buf, out_hbm.at[pl.ds(0, _B * _DIM)], sem).wait()

    def pair_body(p, _):
        for b, (buf, sem) in enumerate(((out_a, sem_a), (out_b, sem_b))):
            ch = 2 * p + b

            @pl.when(p > 0)
            def _():
                drain_copy(ch - 2, buf, sem)

            run_chunk(ch, buf)
            start_copy(ch, buf, sem)
        return 0

    lax.fori_loop(0, _NCH // 2, pair_body, 0)
    for b, (buf, sem) in enumerate(((out_a, sem_a), (out_b, sem_b))):
        drain_copy(_NCH - 2 + b, buf, sem)


def kernel(node_feat, emb_0, emb_1, emb_2, emb_3, emb_4, emb_5, emb_6,
           emb_7, emb_8):
    tab1 = (emb_1[:, None, None, :] + emb_2[None, :, None, :]
            + emb_8[None, None, :, :]).reshape(120, _DIM)
    tab2 = (emb_3[:, None, None, :] + emb_4[None, :, None, :]
            + emb_7[None, None, :, :]).reshape(240, _DIM)
    tab3 = (emb_5[:, None, :] + emb_6[None, :, :]).reshape(36, _DIM)
    tab = jnp.concatenate([emb_0, tab1, tab2, tab3], axis=0)
    tu = jax.lax.bitcast_convert_type(
        tab.astype(jnp.bfloat16), jnp.uint16).astype(jnp.uint32)
    tabp = jax.lax.bitcast_convert_type(
        tu[:, :_D2] | (tu[:, _D2:] << 16), jnp.int32)
    f = node_feat.astype(jnp.int32)
    r0 = f[:, 0]
    r1 = 119 + (f[:, 1] * 12 + f[:, 2]) * 2 + f[:, 8]
    r2 = 239 + (f[:, 3] * 10 + f[:, 4]) * 2 + f[:, 7]
    r3 = 479 + f[:, 5] * 6 + f[:, 6]
    idx = jnp.stack([(r0 * _D2) | ((r1 * _D2) << 16),
                     (r3 * _D2) | ((r2 * _D2) << 16)], axis=1)
    idx = jnp.pad(idx, ((0, _NPAD - _N), (0, 0)))
    idx = idx.T.reshape(_KP, _NW, _CB).transpose(1, 0, 2)

    grid_kernel = functools.partial(
        pl.kernel,
        out_type=jax.ShapeDtypeStruct((_N * _DIM,), jnp.float32),
        mesh=plsc.VectorSubcoreMesh(core_axis_name="c", subcore_axis_name="s"),
        compiler_params=pltpu.CompilerParams(needs_layout_passes=False),
        scratch_types=[
            pltpu.VMEM((_KP, _CB), jnp.int32),
            pltpu.VMEM((_ROWS * _D2,), jnp.int32),
            pltpu.VMEM((_B * _DIM,), jnp.float32),
            pltpu.VMEM((_B * _DIM,), jnp.float32),
            pltpu.SemaphoreType.DMA,
            pltpu.SemaphoreType.DMA,
        ],
    )
    out = grid_kernel(_sc_body)(idx, tabp.reshape(-1))
    return out.reshape(_N, _DIM)
